# baseline (speedup 1.0000x reference)
import functools

import jax
import jax.numpy as jnp
from jax.experimental import pallas as pl
from jax.experimental.pallas import tpu as pltpu

M, D = 131072, 128
TO_ADD_CONST = 1.0
BM = 2048


def _body(idx_ref, x_ref, o_ref):
    o_ref[...] = x_ref[...]
    i = pl.program_id(0)
    idx = idx_ref[0]
    base = i * BM

    @pl.when((idx >= base) & (idx < base + BM))
    def _():
        r = idx - base
        o_ref[pl.ds(r, 1), :] = x_ref[pl.ds(r, 1), :] + TO_ADD_CONST


@functools.partial(jax.jit, static_argnames=())
def _run(tensor, idx_arr):
    grid_spec = pltpu.PrefetchScalarGridSpec(
        num_scalar_prefetch=1,
        grid=(M // BM,),
        in_specs=[pl.BlockSpec((BM, D), lambda i, idx: (i, 0))],
        out_specs=pl.BlockSpec((BM, D), lambda i, idx: (i, 0)),
    )
    return pl.pallas_call(
        _body,
        grid_spec=grid_spec,
        out_shape=jax.ShapeDtypeStruct((M, D), jnp.float32),
    )(idx_arr, tensor)


def kernel(tensor, slice_index, related_index):
    idx_arr = jnp.asarray(slice_index, dtype=jnp.int32).reshape((1,))
    out = _run(tensor, idx_arr)
    return (out, slice_index, related_index)
